# baseline (speedup 1.0000x reference)
import functools
import jax
import jax.numpy as jnp
from jax import lax
from jax.experimental import pallas as pl
from jax.experimental.pallas import tpu as pltpu
from jax.experimental.pallas import tpu_sc as plsc

_N = 10000
_E = 320000
_XD = 128
_HD = 128
_ZD = 64
_HH = 64

_NC = 2
_NS = 16
_NW = _NC * _NS
_EPW = _E // _NW
_CH = 80
_EPWP = 10000
_NCH = _EPWP // _CH
_K = 5
_NP = 10112
_NPS = _NP // _NS
_ZR = 128
_STAGE = [(o, min(_ZR, _NPS - o)) for o in range(0, _NPS, _ZR)]

_BN = 2000
_GRID = _N // _BN


def _fill_vmem(ref, nrows, ncols, value):
    vec = jnp.full((16,), value, jnp.float32)

    def row(r, _):
        def col(j, _):
            ref[r, pl.ds(j * 16, 16)] = vec
            return 0
        return lax.fori_loop(0, ncols // 16, col, 0)

    lax.fori_loop(0, nrows, row, 0)


def _make_segsum(n_tables, with_count):
    outs = []
    if n_tables:
        outs.append(jax.ShapeDtypeStruct((_NC, n_tables, 2, _NP, _HH),
                                         jnp.float32))
    if with_count:
        outs.append(jax.ShapeDtypeStruct((_NC, _NP, 16), jnp.float32))
    scratch = [
        pltpu.VMEM((_NCH, _CH), jnp.int32),
        pltpu.VMEM((_NCH, _CH), jnp.int32),
        pltpu.SemaphoreType.DMA,
        pltpu.SemaphoreType.DMA,
    ]
    if n_tables:
        scratch += [
            pltpu.VMEM((_K, _CH, _HH), jnp.float32),
            pltpu.VMEM((_ZR, _HH), jnp.float32),
            pltpu.VMEM_SHARED((_NP, _HH), jnp.float32),
        ]
    if with_count:
        scratch += [
            pltpu.VMEM((_CH, 16), jnp.float32),
            pltpu.VMEM((_ZR, 16), jnp.float32),
            pltpu.VMEM_SHARED((_NP, 16), jnp.float32),
        ]
    mesh = plsc.VectorSubcoreMesh(core_axis_name="c", subcore_axis_name="s",
                                  num_cores=_NC, num_subcores=_NS)

    @functools.partial(pl.kernel, out_type=tuple(outs), mesh=mesh,
                       scratch_types=scratch,
                       compiler_params=pltpu.CompilerParams(
                           use_tc_tiling_on_sc=False))
    def seg(*refs):
        if n_tables:
            table = refs[0]
            src3, dst3 = refs[1], refs[2]
            out = refs[3]
            o = 4
        else:
            src3, dst3 = refs[0], refs[1]
            o = 2
        if with_count:
            cnt_out = refs[o]
            o += 1
        src_v, dst_v, gsem, ssem = refs[o:o + 4]
        o += 4
        if n_tables:
            rows_v, zbuf, acc = refs[o:o + 3]
            o += 3
        if with_count:
            ones_v, cbuf, cacc = refs[o:o + 3]
        c = lax.axis_index("c")
        s = lax.axis_index("s")
        wid = s * _NC + c
        r0 = s * _NPS

        if n_tables:
            _fill_vmem(zbuf, _ZR, _HH, 0.0)
            pltpu.sync_copy(src3.at[wid], src_v)
        pltpu.sync_copy(dst3.at[wid], dst_v)

        def half_pass(tab, out_slice):
            for st, sz in _STAGE:
                pltpu.sync_copy(zbuf.at[pl.ds(0, sz)],
                                acc.at[pl.ds(r0 + st, sz)])
            plsc.subcore_barrier()

            pltpu.async_copy(tab.at[src_v.at[0]], rows_v.at[0], gsem)
            pltpu.async_copy(tab.at[src_v.at[1]], rows_v.at[1], gsem)

            def chunk(j, _):
                @pl.when(j >= _K - 2)
                def _():
                    pltpu.make_async_copy(rows_v.at[0], acc.at[dst_v.at[0]],
                                          ssem).wait()

                @pl.when(j + 2 < _NCH)
                def _():
                    pltpu.async_copy(tab.at[src_v.at[j + 2]],
                                     rows_v.at[(j + 2) % _K], gsem)

                pltpu.make_async_copy(tab.at[src_v.at[j]],
                                      rows_v.at[j % _K], gsem).wait()
                pltpu.async_copy(rows_v.at[j % _K], acc.at[dst_v.at[j]],
                                 ssem, add=True)
                return 0

            lax.fori_loop(0, _NCH, chunk, 0)
            for _tail in range(_K - 2):
                pltpu.make_async_copy(rows_v.at[0], acc.at[dst_v.at[0]],
                                      ssem).wait()
            plsc.subcore_barrier()

            for st, sz in _STAGE:
                pltpu.sync_copy(acc.at[pl.ds(r0 + st, sz)],
                                zbuf.at[pl.ds(0, sz)])
                pltpu.sync_copy(zbuf.at[pl.ds(0, sz)],
                                out_slice.at[pl.ds(r0 + st, sz)])
            _fill_vmem(zbuf, _ZR, _HH, 0.0)
            plsc.subcore_barrier()

        if n_tables:
            def pass_body(p, _):
                half_pass(table.at[p // 2].at[p % 2],
                          out.at[c].at[p // 2].at[p % 2])
                return 0

            lax.fori_loop(0, 2 * n_tables, pass_body, 0)

        if with_count:
            _fill_vmem(cbuf, _ZR, 16, 0.0)
            _fill_vmem(ones_v, _CH, 16, 1.0)
            for st, sz in _STAGE:
                pltpu.sync_copy(cbuf.at[pl.ds(0, sz)],
                                cacc.at[pl.ds(r0 + st, sz)])
            plsc.subcore_barrier()

            def cgroup(j, _):
                pltpu.sync_copy(ones_v, cacc.at[dst_v.at[j]], add=True)
                return 0

            lax.fori_loop(0, _NCH, cgroup, 0)
            plsc.subcore_barrier()
            for st, sz in _STAGE:
                pltpu.sync_copy(cacc.at[pl.ds(r0 + st, sz)],
                                cbuf.at[pl.ds(0, sz)])
                pltpu.sync_copy(cbuf.at[pl.ds(0, sz)],
                                cnt_out.at[c].at[pl.ds(r0 + st, sz)])

    return seg


@functools.lru_cache(maxsize=None)
def _get_segsum(n_tables=1, with_count=False):
    return _make_segsum(n_tables, with_count)


def _row_spec(d):
    return pl.BlockSpec((_BN, d), lambda i: (i, 0))


def _split_spec():
    return pl.BlockSpec((2, _BN, _HH), lambda i: (0, i, 0))


def _part_spec():
    return pl.BlockSpec((_NC, 2, _BN, _HH), lambda i: (0, 0, i, 0))


def _cnt_spec():
    return pl.BlockSpec((_NC, _BN, 16), lambda i: (0, i, 0))


def _w_spec(r, c):
    return pl.BlockSpec((r, c), lambda i: (0, 0))


def _split_out(d2):
    return jax.ShapeDtypeStruct((2, _N, _HH), jnp.float32)


def _invc(cntp_ref):
    cnt = cntp_ref[0, :, 0:1] + cntp_ref[1, :, 0:1]
    return 1.0 / jnp.maximum(cnt, 1.0)


def _agg(p_ref, ic):
    return jnp.concatenate([p_ref[0, 0] + p_ref[1, 0],
                            p_ref[0, 1] + p_ref[1, 1]], axis=1) * ic


def _cat(sp_ref):
    return jnp.concatenate([sp_ref[0], sp_ref[1]], axis=1)


def _store_split(out_ref, val):
    out_ref[0] = val[:, :_HH]
    out_ref[1] = val[:, _HH:]


def _tc1_body(x, h0s, diff, Wx, bx, Wph, wpd, bp, Wpm, bpm, Wps, bps,
              phix_o, pm_o, ps_o):
    _store_split(phix_o, jax.nn.relu(jnp.dot(x[...], Wx[...]) + bx[...]))
    prior = jax.nn.relu(jnp.dot(_cat(h0s), Wph[...]) + diff[...] * wpd[...]
                        + bp[...])
    pm_o[...] = jnp.dot(prior, Wpm[...]) + bpm[...]
    ps_o[...] = jax.nn.softplus(jnp.dot(prior, Wps[...]) + bps[...])


def _tc2_body(p1x, p1h, cntp, phixs, h0s, Wlt, Wlb, bl, Wrt, Wrb,
              encx_o, agg1x_o, agg1h_o):
    ic = _invc(cntp)
    agg1x = _agg(p1x, ic)
    agg1h = _agg(p1h, ic)
    agg1x_o[...] = agg1x
    agg1h_o[...] = agg1h
    _store_split(encx_o, jax.nn.relu(
        jnp.dot(agg1x, Wlt[...]) + jnp.dot(agg1h, Wlb[...]) + bl[...]
        + jnp.dot(_cat(phixs), Wrt[...]) + jnp.dot(_cat(h0s), Wrb[...])))


def _tc3_body(p2, cntp, encxs, eps, Wml, bml, Wmr, Wsl, bsl, Wsr, Wz, bz,
              mean_o, std_o, z_o, phiz_o):
    agg2 = _agg(p2, _invc(cntp))
    ex = _cat(encxs)
    mean = jnp.dot(agg2, Wml[...]) + bml[...] + jnp.dot(ex, Wmr[...])
    std = jax.nn.softplus(jnp.dot(agg2, Wsl[...]) + bsl[...]
                          + jnp.dot(ex, Wsr[...]))
    zz = eps[...] * std + mean
    mean_o[...] = mean
    std_o[...] = std
    z_o[...] = zz
    phiz_o[...] = jax.nn.relu(jnp.dot(zz, Wz[...]) + bz[...])


def _tc4_body(p3, cntp, agg1x, agg1h, phixs, h0s, phiz,
              gxWl0, gxbl0, gxWr0, gxWl1, gxbl1, gxWr1, gxWl2, gxbl2, gxWr2,
              ghWl0, ghbl0, ghWr0, ghWl1, ghbl1, ghWr1, ghbl2,
              zg_o, rh_o, pre_o):
    agg3 = _agg(p3, _invc(cntp))
    aggx = jnp.concatenate([agg1x[...], agg3], axis=1)
    aggh = agg1h[...]
    h0v = _cat(h0s)
    rnn = jnp.concatenate([_cat(phixs), phiz[...]], axis=1)
    zg = jax.nn.sigmoid(jnp.dot(aggx, gxWl0[...]) + gxbl0[...]
                        + jnp.dot(rnn, gxWr0[...])
                        + jnp.dot(aggh, ghWl0[...]) + ghbl0[...]
                        + jnp.dot(h0v, ghWr0[...]))
    rg = jax.nn.sigmoid(jnp.dot(aggx, gxWl1[...]) + gxbl1[...]
                        + jnp.dot(rnn, gxWr1[...])
                        + jnp.dot(aggh, ghWl1[...]) + ghbl1[...]
                        + jnp.dot(h0v, ghWr1[...]))
    zg_o[...] = zg
    _store_split(rh_o, rg * h0v)
    pre_o[...] = (jnp.dot(aggx, gxWl2[...]) + gxbl2[...]
                  + jnp.dot(rnn, gxWr2[...]) + ghbl2[...])


def _tc5_body(p4, cntp, pre, rhs, zg, h0s, ghWl2, ghWr2, out_o):
    agg4 = _agg(p4, _invc(cntp))
    ht = jnp.tanh(pre[...] + jnp.dot(agg4, ghWl2[...])
                  + jnp.dot(_cat(rhs), ghWr2[...]))
    z = zg[...]
    out_o[...] = z * _cat(h0s) + (1.0 - z) * ht


def kernel(x, h, diff, edge_index, W_phi_x, b_phi_x, enc_Wl, enc_bl, enc_Wr,
           encm_Wl, encm_bl, encm_Wr, encs_Wl, encs_bl, encs_Wr,
           W_prior, b_prior, W_pm, b_pm, W_ps, b_ps, W_phi_z, b_phi_z,
           gx_Wl, gx_bl, gx_Wr, gh_Wl, gh_bl, gh_Wr):
    h0 = h[0]
    h0s = jnp.stack([h0[:, :_HH], h0[:, _HH:]])
    pad = _EPWP - _EPW
    src3 = jnp.concatenate(
        [edge_index[0].astype(jnp.int32).reshape(_NW, _EPW),
         jnp.zeros((_NW, pad), jnp.int32)], axis=1).reshape(_NW, _NCH, _CH)
    pad_dst = _N + (jnp.arange(pad, dtype=jnp.int32) % (_NP - _N))
    dst3 = jnp.concatenate(
        [edge_index[1].astype(jnp.int32).reshape(_NW, _EPW),
         jnp.broadcast_to(pad_dst, (_NW, pad))], axis=1).reshape(
             _NW, _NCH, _CH)
    eps1 = jax.random.normal(jax.random.key(7), (_N, _ZD), dtype=jnp.float32)
    r2 = lambda b: b.reshape(1, -1)
    segsum = _get_segsum()
    split_shape = jax.ShapeDtypeStruct((2, _N, _HH), jnp.float32)
    row_shape = lambda d: jax.ShapeDtypeStruct((_N, d), jnp.float32)

    phixs, prior_mean, prior_std = pl.pallas_call(
        _tc1_body,
        grid=(_GRID,),
        in_specs=[_row_spec(_XD), _split_spec(), _row_spec(1),
                  _w_spec(_XD, _HD), _w_spec(1, _HD),
                  _w_spec(_HD, _HD), _w_spec(1, _HD), _w_spec(1, _HD),
                  _w_spec(_HD, _ZD), _w_spec(1, _ZD),
                  _w_spec(_HD, _ZD), _w_spec(1, _ZD)],
        out_specs=[_split_spec(), _row_spec(_ZD), _row_spec(_ZD)],
        out_shape=[split_shape, row_shape(_ZD), row_shape(_ZD)],
    )(x, h0s, diff, W_phi_x, r2(b_phi_x), W_prior[:_HD], r2(W_prior[_HD]),
      r2(b_prior), W_pm, r2(b_pm), W_ps, r2(b_ps))

    p1x = segsum(phixs[None], src3, dst3)[0][:, 0]
    p1h = segsum(h0s[None], src3, dst3)[0][:, 0]
    (cntp,) = _get_segsum(0, True)(src3, dst3)

    encxs, agg1x, agg1h = pl.pallas_call(
        _tc2_body,
        grid=(_GRID,),
        in_specs=[_part_spec(), _part_spec(), _cnt_spec(),
                  _split_spec(), _split_spec(),
                  _w_spec(_HD, _HD), _w_spec(_HD, _HD), _w_spec(1, _HD),
                  _w_spec(_HD, _HD), _w_spec(_HD, _HD)],
        out_specs=[_split_spec(), _row_spec(_HD), _row_spec(_HD)],
        out_shape=[split_shape, row_shape(_HD), row_shape(_HD)],
    )(p1x, p1h, cntp, phixs, h0s,
      enc_Wl[:_HD], enc_Wl[_HD:], r2(enc_bl), enc_Wr[:_HD], enc_Wr[_HD:])

    p2 = segsum(encxs[None], src3, dst3)[0][:, 0]

    enc_x_mean, enc_x_std, z, phiz = pl.pallas_call(
        _tc3_body,
        grid=(_GRID,),
        in_specs=[_part_spec(), _cnt_spec(), _split_spec(),
                  _row_spec(_ZD),
                  _w_spec(_HD, _ZD), _w_spec(1, _ZD), _w_spec(_HD, _ZD),
                  _w_spec(_HD, _ZD), _w_spec(1, _ZD), _w_spec(_HD, _ZD),
                  _w_spec(_ZD, _HD), _w_spec(1, _HD)],
        out_specs=[_row_spec(_ZD), _row_spec(_ZD), _row_spec(_ZD),
                   _row_spec(_HD)],
        out_shape=[row_shape(_ZD), row_shape(_ZD), row_shape(_ZD),
                   row_shape(_HD)],
    )(p2, cntp, encxs, eps1, encm_Wl, r2(encm_bl), encm_Wr,
      encs_Wl, r2(encs_bl), encs_Wr, W_phi_z, r2(b_phi_z))

    phizs = jnp.stack([phiz[:, :_HH], phiz[:, _HH:]])
    p3 = segsum(phizs[None], src3, dst3)[0][:, 0]

    zg, rhs, pre = pl.pallas_call(
        _tc4_body,
        grid=(_GRID,),
        in_specs=[_part_spec(), _cnt_spec(), _row_spec(_HD),
                  _row_spec(_HD), _split_spec(), _split_spec(),
                  _row_spec(_HD)]
                 + [_w_spec(2 * _HD, _HD), _w_spec(1, _HD),
                    _w_spec(2 * _HD, _HD)] * 3
                 + [_w_spec(_HD, _HD), _w_spec(1, _HD),
                    _w_spec(_HD, _HD)] * 2
                 + [_w_spec(1, _HD)],
        out_specs=[_row_spec(_HD), _split_spec(), _row_spec(_HD)],
        out_shape=[row_shape(_HD), split_shape, row_shape(_HD)],
    )(p3, cntp, agg1x, agg1h, phixs, h0s, phiz,
      gx_Wl[0], r2(gx_bl[0]), gx_Wr[0],
      gx_Wl[1], r2(gx_bl[1]), gx_Wr[1],
      gx_Wl[2], r2(gx_bl[2]), gx_Wr[2],
      gh_Wl[0], r2(gh_bl[0]), gh_Wr[0],
      gh_Wl[1], r2(gh_bl[1]), gh_Wr[1],
      r2(gh_bl[2]))

    p4 = segsum(rhs[None], src3, dst3)[0][:, 0]

    out = pl.pallas_call(
        _tc5_body,
        grid=(_GRID,),
        in_specs=[_part_spec(), _cnt_spec(), _row_spec(_HD),
                  _split_spec(), _row_spec(_HD), _split_spec(),
                  _w_spec(_HD, _HD), _w_spec(_HD, _HD)],
        out_specs=[_row_spec(_HD)],
        out_shape=[row_shape(_HD)],
    )(p4, cntp, pre, rhs, zg, h0s, gh_Wl[2], gh_Wr[2])[0]

    return (prior_mean, prior_std, enc_x_mean, enc_x_std, z, out[None])


# speedup vs baseline: 1.0226x; 1.0226x over previous
import functools
import jax
import jax.numpy as jnp
from jax import lax
from jax.experimental import pallas as pl
from jax.experimental.pallas import tpu as pltpu
from jax.experimental.pallas import tpu_sc as plsc

_N = 10000
_E = 320000
_XD = 128
_HD = 128
_ZD = 64
_HH = 64

_NC = 2
_NS = 16
_NW = _NC * _NS
_EPW = _E // _NW
_CH = 80
_EPWP = 10000
_NCH = _EPWP // _CH
_K = 4
_NP = 10112
_NPS = _NP // _NS
_ZR = 128
_STAGE = [(o, min(_ZR, _NPS - o)) for o in range(0, _NPS, _ZR)]

_BN = 2000
_GRID = _N // _BN


def _fill_vmem(ref, nrows, ncols, value):
    vec = jnp.full((16,), value, jnp.float32)

    def row(r, _):
        def col(j, _):
            ref[r, pl.ds(j * 16, 16)] = vec
            return 0
        return lax.fori_loop(0, ncols // 16, col, 0)

    lax.fori_loop(0, nrows, row, 0)


def _make_segsum(n_tables, with_count):
    outs = []
    if n_tables:
        outs.append(jax.ShapeDtypeStruct((_NC, n_tables, 2, _NP, _HH),
                                         jnp.float32))
    if with_count:
        outs.append(jax.ShapeDtypeStruct((_NC, _NP, 16), jnp.float32))
    scratch = [
        pltpu.VMEM((_NCH, _CH), jnp.int32),
        pltpu.VMEM((_NCH, _CH), jnp.int32),
        pltpu.SemaphoreType.DMA,
        pltpu.SemaphoreType.DMA,
    ]
    if n_tables:
        scratch += [
            pltpu.VMEM((_K, _CH, _HH), jnp.float32),
            pltpu.VMEM((_ZR, _HH), jnp.float32),
            pltpu.VMEM_SHARED((_NP, _HH), jnp.float32),
        ]
    if with_count:
        scratch += [
            pltpu.VMEM((_CH, 16), jnp.float32),
            pltpu.VMEM((_ZR, 16), jnp.float32),
            pltpu.VMEM_SHARED((_NP, 16), jnp.float32),
        ]
    mesh = plsc.VectorSubcoreMesh(core_axis_name="c", subcore_axis_name="s",
                                  num_cores=_NC, num_subcores=_NS)

    @functools.partial(pl.kernel, out_type=tuple(outs), mesh=mesh,
                       scratch_types=scratch,
                       compiler_params=pltpu.CompilerParams(
                           use_tc_tiling_on_sc=False))
    def seg(*refs):
        if n_tables:
            table = refs[0]
            src3, dst3 = refs[1], refs[2]
            out = refs[3]
            o = 4
        else:
            src3, dst3 = refs[0], refs[1]
            o = 2
        if with_count:
            cnt_out = refs[o]
            o += 1
        src_v, dst_v, gsem, ssem = refs[o:o + 4]
        o += 4
        if n_tables:
            rows_v, zbuf, acc = refs[o:o + 3]
            o += 3
        if with_count:
            ones_v, cbuf, cacc = refs[o:o + 3]
        c = lax.axis_index("c")
        s = lax.axis_index("s")
        wid = s * _NC + c
        r0 = s * _NPS

        if n_tables:
            _fill_vmem(zbuf, _ZR, _HH, 0.0)
            pltpu.sync_copy(src3.at[wid], src_v)
        pltpu.sync_copy(dst3.at[wid], dst_v)

        def half_pass(tab, out_slice):
            for st, sz in _STAGE:
                pltpu.sync_copy(zbuf.at[pl.ds(0, sz)],
                                acc.at[pl.ds(r0 + st, sz)])
            plsc.subcore_barrier()

            pltpu.async_copy(tab.at[src_v.at[0]], rows_v.at[0], gsem)
            pltpu.async_copy(tab.at[src_v.at[1]], rows_v.at[1], gsem)

            def chunk(j, _):
                @pl.when(j >= _K - 2)
                def _():
                    pltpu.make_async_copy(rows_v.at[0], acc.at[dst_v.at[0]],
                                          ssem).wait()

                @pl.when(j + 2 < _NCH)
                def _():
                    pltpu.async_copy(tab.at[src_v.at[j + 2]],
                                     rows_v.at[(j + 2) % _K], gsem)

                pltpu.make_async_copy(tab.at[src_v.at[j]],
                                      rows_v.at[j % _K], gsem).wait()
                pltpu.async_copy(rows_v.at[j % _K], acc.at[dst_v.at[j]],
                                 ssem, add=True)
                return 0

            lax.fori_loop(0, _NCH, chunk, 0)
            for _tail in range(_K - 2):
                pltpu.make_async_copy(rows_v.at[0], acc.at[dst_v.at[0]],
                                      ssem).wait()
            plsc.subcore_barrier()

            for st, sz in _STAGE:
                pltpu.sync_copy(acc.at[pl.ds(r0 + st, sz)],
                                zbuf.at[pl.ds(0, sz)])
                pltpu.sync_copy(zbuf.at[pl.ds(0, sz)],
                                out_slice.at[pl.ds(r0 + st, sz)])
            _fill_vmem(zbuf, _ZR, _HH, 0.0)
            plsc.subcore_barrier()

        if n_tables:
            def pass_body(p, _):
                half_pass(table.at[p // 2].at[p % 2],
                          out.at[c].at[p // 2].at[p % 2])
                return 0

            lax.fori_loop(0, 2 * n_tables, pass_body, 0)

        if with_count:
            _fill_vmem(cbuf, _ZR, 16, 0.0)
            _fill_vmem(ones_v, _CH, 16, 1.0)
            for st, sz in _STAGE:
                pltpu.sync_copy(cbuf.at[pl.ds(0, sz)],
                                cacc.at[pl.ds(r0 + st, sz)])
            plsc.subcore_barrier()

            def cgroup(j, _):
                @pl.when(j >= 2)
                def _():
                    pltpu.make_async_copy(ones_v, cacc.at[dst_v.at[0]],
                                          ssem).wait()

                pltpu.async_copy(ones_v, cacc.at[dst_v.at[j]], ssem, add=True)
                return 0

            lax.fori_loop(0, _NCH, cgroup, 0)
            for _tail in range(2):
                pltpu.make_async_copy(ones_v, cacc.at[dst_v.at[0]],
                                      ssem).wait()
            plsc.subcore_barrier()
            for st, sz in _STAGE:
                pltpu.sync_copy(cacc.at[pl.ds(r0 + st, sz)],
                                cbuf.at[pl.ds(0, sz)])
                pltpu.sync_copy(cbuf.at[pl.ds(0, sz)],
                                cnt_out.at[c].at[pl.ds(r0 + st, sz)])

    return seg


@functools.lru_cache(maxsize=None)
def _get_segsum(n_tables=1, with_count=False):
    return _make_segsum(n_tables, with_count)


def _row_spec(d):
    return pl.BlockSpec((_BN, d), lambda i: (i, 0))


def _split_spec():
    return pl.BlockSpec((2, _BN, _HH), lambda i: (0, i, 0))


def _part_spec():
    return pl.BlockSpec((_NC, 2, _BN, _HH), lambda i: (0, 0, i, 0))


def _cnt_spec():
    return pl.BlockSpec((_NC, _BN, 16), lambda i: (0, i, 0))


def _w_spec(r, c):
    return pl.BlockSpec((r, c), lambda i: (0, 0))


def _split_out(d2):
    return jax.ShapeDtypeStruct((2, _N, _HH), jnp.float32)


def _invc(cntp_ref):
    cnt = cntp_ref[0, :, 0:1] + cntp_ref[1, :, 0:1]
    return 1.0 / jnp.maximum(cnt, 1.0)


def _agg(p_ref, ic):
    return jnp.concatenate([p_ref[0, 0] + p_ref[1, 0],
                            p_ref[0, 1] + p_ref[1, 1]], axis=1) * ic


def _cat(sp_ref):
    return jnp.concatenate([sp_ref[0], sp_ref[1]], axis=1)


def _store_split(out_ref, val):
    out_ref[0] = val[:, :_HH]
    out_ref[1] = val[:, _HH:]


def _tc1_body(x, h0s, diff, Wx, bx, Wph, wpd, bp, Wpm, bpm, Wps, bps,
              phix_o, pm_o, ps_o):
    _store_split(phix_o, jax.nn.relu(jnp.dot(x[...], Wx[...]) + bx[...]))
    prior = jax.nn.relu(jnp.dot(_cat(h0s), Wph[...]) + diff[...] * wpd[...]
                        + bp[...])
    pm_o[...] = jnp.dot(prior, Wpm[...]) + bpm[...]
    ps_o[...] = jax.nn.softplus(jnp.dot(prior, Wps[...]) + bps[...])


def _tc2_body(p1x, p1h, cntp, phixs, h0s, Wlt, Wlb, bl, Wrt, Wrb,
              encx_o, agg1x_o, agg1h_o):
    ic = _invc(cntp)
    agg1x = _agg(p1x, ic)
    agg1h = _agg(p1h, ic)
    agg1x_o[...] = agg1x
    agg1h_o[...] = agg1h
    _store_split(encx_o, jax.nn.relu(
        jnp.dot(agg1x, Wlt[...]) + jnp.dot(agg1h, Wlb[...]) + bl[...]
        + jnp.dot(_cat(phixs), Wrt[...]) + jnp.dot(_cat(h0s), Wrb[...])))


def _tc3_body(p2, cntp, encxs, eps, Wml, bml, Wmr, Wsl, bsl, Wsr, Wz, bz,
              mean_o, std_o, z_o, phiz_o):
    agg2 = _agg(p2, _invc(cntp))
    ex = _cat(encxs)
    mean = jnp.dot(agg2, Wml[...]) + bml[...] + jnp.dot(ex, Wmr[...])
    std = jax.nn.softplus(jnp.dot(agg2, Wsl[...]) + bsl[...]
                          + jnp.dot(ex, Wsr[...]))
    zz = eps[...] * std + mean
    mean_o[...] = mean
    std_o[...] = std
    z_o[...] = zz
    phiz_o[...] = jax.nn.relu(jnp.dot(zz, Wz[...]) + bz[...])


def _tc4_body(p3, cntp, agg1x, agg1h, phixs, h0s, phiz,
              gxWl0, gxbl0, gxWr0, gxWl1, gxbl1, gxWr1, gxWl2, gxbl2, gxWr2,
              ghWl0, ghbl0, ghWr0, ghWl1, ghbl1, ghWr1, ghbl2,
              zg_o, rh_o, pre_o):
    agg3 = _agg(p3, _invc(cntp))
    aggx = jnp.concatenate([agg1x[...], agg3], axis=1)
    aggh = agg1h[...]
    h0v = _cat(h0s)
    rnn = jnp.concatenate([_cat(phixs), phiz[...]], axis=1)
    zg = jax.nn.sigmoid(jnp.dot(aggx, gxWl0[...]) + gxbl0[...]
                        + jnp.dot(rnn, gxWr0[...])
                        + jnp.dot(aggh, ghWl0[...]) + ghbl0[...]
                        + jnp.dot(h0v, ghWr0[...]))
    rg = jax.nn.sigmoid(jnp.dot(aggx, gxWl1[...]) + gxbl1[...]
                        + jnp.dot(rnn, gxWr1[...])
                        + jnp.dot(aggh, ghWl1[...]) + ghbl1[...]
                        + jnp.dot(h0v, ghWr1[...]))
    zg_o[...] = zg
    _store_split(rh_o, rg * h0v)
    pre_o[...] = (jnp.dot(aggx, gxWl2[...]) + gxbl2[...]
                  + jnp.dot(rnn, gxWr2[...]) + ghbl2[...])


def _tc5_body(p4, cntp, pre, rhs, zg, h0s, ghWl2, ghWr2, out_o):
    agg4 = _agg(p4, _invc(cntp))
    ht = jnp.tanh(pre[...] + jnp.dot(agg4, ghWl2[...])
                  + jnp.dot(_cat(rhs), ghWr2[...]))
    z = zg[...]
    out_o[...] = z * _cat(h0s) + (1.0 - z) * ht


def kernel(x, h, diff, edge_index, W_phi_x, b_phi_x, enc_Wl, enc_bl, enc_Wr,
           encm_Wl, encm_bl, encm_Wr, encs_Wl, encs_bl, encs_Wr,
           W_prior, b_prior, W_pm, b_pm, W_ps, b_ps, W_phi_z, b_phi_z,
           gx_Wl, gx_bl, gx_Wr, gh_Wl, gh_bl, gh_Wr):
    h0 = h[0]
    h0s = jnp.stack([h0[:, :_HH], h0[:, _HH:]])
    pad = _EPWP - _EPW
    src3 = jnp.concatenate(
        [edge_index[0].astype(jnp.int32).reshape(_NW, _EPW),
         jnp.zeros((_NW, pad), jnp.int32)], axis=1).reshape(_NW, _NCH, _CH)
    pad_dst = _N + (jnp.arange(pad, dtype=jnp.int32) % (_NP - _N))
    dst3 = jnp.concatenate(
        [edge_index[1].astype(jnp.int32).reshape(_NW, _EPW),
         jnp.broadcast_to(pad_dst, (_NW, pad))], axis=1).reshape(
             _NW, _NCH, _CH)
    eps1 = jax.random.normal(jax.random.key(7), (_N, _ZD), dtype=jnp.float32)
    r2 = lambda b: b.reshape(1, -1)
    segsum = _get_segsum()
    split_shape = jax.ShapeDtypeStruct((2, _N, _HH), jnp.float32)
    row_shape = lambda d: jax.ShapeDtypeStruct((_N, d), jnp.float32)

    phixs, prior_mean, prior_std = pl.pallas_call(
        _tc1_body,
        grid=(_GRID,),
        in_specs=[_row_spec(_XD), _split_spec(), _row_spec(1),
                  _w_spec(_XD, _HD), _w_spec(1, _HD),
                  _w_spec(_HD, _HD), _w_spec(1, _HD), _w_spec(1, _HD),
                  _w_spec(_HD, _ZD), _w_spec(1, _ZD),
                  _w_spec(_HD, _ZD), _w_spec(1, _ZD)],
        out_specs=[_split_spec(), _row_spec(_ZD), _row_spec(_ZD)],
        out_shape=[split_shape, row_shape(_ZD), row_shape(_ZD)],
    )(x, h0s, diff, W_phi_x, r2(b_phi_x), W_prior[:_HD], r2(W_prior[_HD]),
      r2(b_prior), W_pm, r2(b_pm), W_ps, r2(b_ps))

    p1x = segsum(phixs[None], src3, dst3)[0][:, 0]
    p1h = segsum(h0s[None], src3, dst3)[0][:, 0]
    (cntp,) = _get_segsum(0, True)(src3, dst3)

    encxs, agg1x, agg1h = pl.pallas_call(
        _tc2_body,
        grid=(_GRID,),
        in_specs=[_part_spec(), _part_spec(), _cnt_spec(),
                  _split_spec(), _split_spec(),
                  _w_spec(_HD, _HD), _w_spec(_HD, _HD), _w_spec(1, _HD),
                  _w_spec(_HD, _HD), _w_spec(_HD, _HD)],
        out_specs=[_split_spec(), _row_spec(_HD), _row_spec(_HD)],
        out_shape=[split_shape, row_shape(_HD), row_shape(_HD)],
    )(p1x, p1h, cntp, phixs, h0s,
      enc_Wl[:_HD], enc_Wl[_HD:], r2(enc_bl), enc_Wr[:_HD], enc_Wr[_HD:])

    p2 = segsum(encxs[None], src3, dst3)[0][:, 0]

    enc_x_mean, enc_x_std, z, phiz = pl.pallas_call(
        _tc3_body,
        grid=(_GRID,),
        in_specs=[_part_spec(), _cnt_spec(), _split_spec(),
                  _row_spec(_ZD),
                  _w_spec(_HD, _ZD), _w_spec(1, _ZD), _w_spec(_HD, _ZD),
                  _w_spec(_HD, _ZD), _w_spec(1, _ZD), _w_spec(_HD, _ZD),
                  _w_spec(_ZD, _HD), _w_spec(1, _HD)],
        out_specs=[_row_spec(_ZD), _row_spec(_ZD), _row_spec(_ZD),
                   _row_spec(_HD)],
        out_shape=[row_shape(_ZD), row_shape(_ZD), row_shape(_ZD),
                   row_shape(_HD)],
    )(p2, cntp, encxs, eps1, encm_Wl, r2(encm_bl), encm_Wr,
      encs_Wl, r2(encs_bl), encs_Wr, W_phi_z, r2(b_phi_z))

    phizs = jnp.stack([phiz[:, :_HH], phiz[:, _HH:]])
    p3 = segsum(phizs[None], src3, dst3)[0][:, 0]

    zg, rhs, pre = pl.pallas_call(
        _tc4_body,
        grid=(_GRID,),
        in_specs=[_part_spec(), _cnt_spec(), _row_spec(_HD),
                  _row_spec(_HD), _split_spec(), _split_spec(),
                  _row_spec(_HD)]
                 + [_w_spec(2 * _HD, _HD), _w_spec(1, _HD),
                    _w_spec(2 * _HD, _HD)] * 3
                 + [_w_spec(_HD, _HD), _w_spec(1, _HD),
                    _w_spec(_HD, _HD)] * 2
                 + [_w_spec(1, _HD)],
        out_specs=[_row_spec(_HD), _split_spec(), _row_spec(_HD)],
        out_shape=[row_shape(_HD), split_shape, row_shape(_HD)],
    )(p3, cntp, agg1x, agg1h, phixs, h0s, phiz,
      gx_Wl[0], r2(gx_bl[0]), gx_Wr[0],
      gx_Wl[1], r2(gx_bl[1]), gx_Wr[1],
      gx_Wl[2], r2(gx_bl[2]), gx_Wr[2],
      gh_Wl[0], r2(gh_bl[0]), gh_Wr[0],
      gh_Wl[1], r2(gh_bl[1]), gh_Wr[1],
      r2(gh_bl[2]))

    p4 = segsum(rhs[None], src3, dst3)[0][:, 0]

    out = pl.pallas_call(
        _tc5_body,
        grid=(_GRID,),
        in_specs=[_part_spec(), _cnt_spec(), _row_spec(_HD),
                  _split_spec(), _row_spec(_HD), _split_spec(),
                  _w_spec(_HD, _HD), _w_spec(_HD, _HD)],
        out_specs=[_row_spec(_HD)],
        out_shape=[row_shape(_HD)],
    )(p4, cntp, pre, rhs, zg, h0s, gh_Wl[2], gh_Wr[2])[0]

    return (prior_mean, prior_std, enc_x_mean, enc_x_std, z, out[None])


# speedup vs baseline: 1.1236x; 1.0988x over previous
import functools
import jax
import jax.numpy as jnp
from jax import lax
from jax.experimental import pallas as pl
from jax.experimental.pallas import tpu as pltpu
from jax.experimental.pallas import tpu_sc as plsc

_N = 10000
_E = 320000
_XD = 128
_HD = 128
_ZD = 64
_HH = 64

_NC = 2
_NS = 16
_EPW = _E // _NS
_CH = 80
_NCH = _EPW // _CH
_K = 4
_NP = 10112
_NPS = _NP // _NS
_ZR = 128
_STAGE = [(o, min(_ZR, _NPS - o)) for o in range(0, _NPS, _ZR)]

_BN = 2000
_GRID = _N // _BN


def _fill_vmem(ref, nrows, ncols, value):
    vec = jnp.full((16,), value, jnp.float32)

    def row(r, _):
        def col(j, _):
            ref[r, pl.ds(j * 16, 16)] = vec
            return 0
        return lax.fori_loop(0, ncols // 16, col, 0)

    lax.fori_loop(0, nrows, row, 0)


def _make_segsum(n_tables, with_count):
    outs = []
    if n_tables:
        outs.append(jax.ShapeDtypeStruct((_NC, n_tables, _NP, _HH),
                                         jnp.float32))
    if with_count:
        outs.append(jax.ShapeDtypeStruct((1, _NP, 16), jnp.float32))
    scratch = [
        pltpu.VMEM((_NCH, _CH), jnp.int32),
        pltpu.VMEM((_NCH, _CH), jnp.int32),
        pltpu.SemaphoreType.DMA,
        pltpu.SemaphoreType.DMA,
    ]
    if n_tables:
        scratch += [
            pltpu.VMEM((_K, _CH, _HH), jnp.float32),
            pltpu.VMEM((_ZR, _HH), jnp.float32),
            pltpu.VMEM_SHARED((_NP, _HH), jnp.float32),
        ]
    if with_count:
        scratch += [
            pltpu.VMEM((_CH, 16), jnp.float32),
            pltpu.VMEM((_ZR, 16), jnp.float32),
            pltpu.VMEM_SHARED((_NP, 16), jnp.float32),
        ]
    mesh = plsc.VectorSubcoreMesh(core_axis_name="c", subcore_axis_name="s",
                                  num_cores=_NC, num_subcores=_NS)

    @functools.partial(pl.kernel, out_type=tuple(outs), mesh=mesh,
                       scratch_types=scratch,
                       compiler_params=pltpu.CompilerParams(
                           use_tc_tiling_on_sc=False))
    def seg(*refs):
        if n_tables:
            table = refs[0]
            src3, dst3 = refs[1], refs[2]
            out = refs[3]
            o = 4
        else:
            src3, dst3 = refs[0], refs[1]
            o = 2
        if with_count:
            cnt_out = refs[o]
            o += 1
        src_v, dst_v, gsem, ssem = refs[o:o + 4]
        o += 4
        if n_tables:
            rows_v, zbuf, acc = refs[o:o + 3]
            o += 3
        if with_count:
            ones_v, cbuf, cacc = refs[o:o + 3]
        c = lax.axis_index("c")
        s = lax.axis_index("s")
        r0 = s * _NPS

        if n_tables:
            _fill_vmem(zbuf, _ZR, _HH, 0.0)
            pltpu.sync_copy(src3.at[s], src_v)
        pltpu.sync_copy(dst3.at[s], dst_v)

        def half_pass(tab, out_slice):
            for st, sz in _STAGE:
                pltpu.sync_copy(zbuf.at[pl.ds(0, sz)],
                                acc.at[pl.ds(r0 + st, sz)])
            plsc.subcore_barrier()

            pltpu.async_copy(tab.at[src_v.at[0]], rows_v.at[0], gsem)
            pltpu.async_copy(tab.at[src_v.at[1]], rows_v.at[1], gsem)

            def chunk(j, _):
                @pl.when(j >= _K - 2)
                def _():
                    pltpu.make_async_copy(rows_v.at[0], acc.at[dst_v.at[0]],
                                          ssem).wait()

                @pl.when(j + 2 < _NCH)
                def _():
                    pltpu.async_copy(tab.at[src_v.at[j + 2]],
                                     rows_v.at[(j + 2) % _K], gsem)

                pltpu.make_async_copy(tab.at[src_v.at[j]],
                                      rows_v.at[j % _K], gsem).wait()
                pltpu.async_copy(rows_v.at[j % _K], acc.at[dst_v.at[j]],
                                 ssem, add=True)
                return 0

            lax.fori_loop(0, _NCH, chunk, 0)
            for _tail in range(_K - 2):
                pltpu.make_async_copy(rows_v.at[0], acc.at[dst_v.at[0]],
                                      ssem).wait()
            plsc.subcore_barrier()

            for st, sz in _STAGE:
                pltpu.sync_copy(acc.at[pl.ds(r0 + st, sz)],
                                zbuf.at[pl.ds(0, sz)])
                pltpu.sync_copy(zbuf.at[pl.ds(0, sz)],
                                out_slice.at[pl.ds(r0 + st, sz)])
            _fill_vmem(zbuf, _ZR, _HH, 0.0)
            plsc.subcore_barrier()

        if n_tables:
            def pass_body(t, _):
                half_pass(table.at[t].at[c], out.at[c].at[t])
                return 0

            lax.fori_loop(0, n_tables, pass_body, 0)

        if with_count:
            _fill_vmem(cbuf, _ZR, 16, 0.0)
            _fill_vmem(ones_v, _CH, 16, 1.0)
            for st, sz in _STAGE:
                pltpu.sync_copy(cbuf.at[pl.ds(0, sz)],
                                cacc.at[pl.ds(r0 + st, sz)])
            plsc.subcore_barrier()

            def cgroup(j, _):
                @pl.when(j >= 2)
                def _():
                    pltpu.make_async_copy(ones_v, cacc.at[dst_v.at[0]],
                                          ssem).wait()

                pltpu.async_copy(ones_v, cacc.at[dst_v.at[j]], ssem, add=True)
                return 0

            lax.fori_loop(0, _NCH, cgroup, 0)
            for _tail in range(2):
                pltpu.make_async_copy(ones_v, cacc.at[dst_v.at[0]],
                                      ssem).wait()
            plsc.subcore_barrier()
            @pl.when(c == 0)
            def _():
                for st, sz in _STAGE:
                    pltpu.sync_copy(cacc.at[pl.ds(r0 + st, sz)],
                                    cbuf.at[pl.ds(0, sz)])
                    pltpu.sync_copy(cbuf.at[pl.ds(0, sz)],
                                    cnt_out.at[0].at[pl.ds(r0 + st, sz)])

    return seg


@functools.lru_cache(maxsize=None)
def _get_segsum(n_tables=1, with_count=False):
    return _make_segsum(n_tables, with_count)


def _row_spec(d):
    return pl.BlockSpec((_BN, d), lambda i: (i, 0))


def _split_spec():
    return pl.BlockSpec((2, _BN, _HH), lambda i: (0, i, 0))


def _part_spec():
    return pl.BlockSpec((_NC, _BN, _HH), lambda i: (0, i, 0))


def _cnt_spec():
    return pl.BlockSpec((1, _BN, 16), lambda i: (0, i, 0))


def _w_spec(r, c):
    return pl.BlockSpec((r, c), lambda i: (0, 0))


def _split_out(d2):
    return jax.ShapeDtypeStruct((2, _N, _HH), jnp.float32)


def _invc(cntp_ref):
    return 1.0 / jnp.maximum(cntp_ref[0, :, 0:1], 1.0)


def _agg(p_ref, ic):
    return jnp.concatenate([p_ref[0], p_ref[1]], axis=1) * ic


def _cat(sp_ref):
    return jnp.concatenate([sp_ref[0], sp_ref[1]], axis=1)


def _store_split(out_ref, val):
    out_ref[0] = val[:, :_HH]
    out_ref[1] = val[:, _HH:]


def _tc1_body(x, h0s, diff, Wx, bx, Wph, wpd, bp, Wpm, bpm, Wps, bps,
              phix_o, pm_o, ps_o):
    _store_split(phix_o, jax.nn.relu(jnp.dot(x[...], Wx[...]) + bx[...]))
    prior = jax.nn.relu(jnp.dot(_cat(h0s), Wph[...]) + diff[...] * wpd[...]
                        + bp[...])
    pm_o[...] = jnp.dot(prior, Wpm[...]) + bpm[...]
    ps_o[...] = jax.nn.softplus(jnp.dot(prior, Wps[...]) + bps[...])


def _tc2_body(p1x, p1h, cntp, phixs, h0s, Wlt, Wlb, bl, Wrt, Wrb,
              encx_o, agg1x_o, agg1h_o):
    ic = _invc(cntp)
    agg1x = _agg(p1x, ic)
    agg1h = _agg(p1h, ic)
    agg1x_o[...] = agg1x
    agg1h_o[...] = agg1h
    _store_split(encx_o, jax.nn.relu(
        jnp.dot(agg1x, Wlt[...]) + jnp.dot(agg1h, Wlb[...]) + bl[...]
        + jnp.dot(_cat(phixs), Wrt[...]) + jnp.dot(_cat(h0s), Wrb[...])))


def _tc3_body(p2, cntp, encxs, eps, Wml, bml, Wmr, Wsl, bsl, Wsr, Wz, bz,
              mean_o, std_o, z_o, phiz_o):
    agg2 = _agg(p2, _invc(cntp))
    ex = _cat(encxs)
    mean = jnp.dot(agg2, Wml[...]) + bml[...] + jnp.dot(ex, Wmr[...])
    std = jax.nn.softplus(jnp.dot(agg2, Wsl[...]) + bsl[...]
                          + jnp.dot(ex, Wsr[...]))
    zz = eps[...] * std + mean
    mean_o[...] = mean
    std_o[...] = std
    z_o[...] = zz
    phiz_o[...] = jax.nn.relu(jnp.dot(zz, Wz[...]) + bz[...])


def _tc4_body(p3, cntp, agg1x, agg1h, phixs, h0s, phiz,
              gxWl0, gxbl0, gxWr0, gxWl1, gxbl1, gxWr1, gxWl2, gxbl2, gxWr2,
              ghWl0, ghbl0, ghWr0, ghWl1, ghbl1, ghWr1, ghbl2,
              zg_o, rh_o, pre_o):
    agg3 = _agg(p3, _invc(cntp))
    aggx = jnp.concatenate([agg1x[...], agg3], axis=1)
    aggh = agg1h[...]
    h0v = _cat(h0s)
    rnn = jnp.concatenate([_cat(phixs), phiz[...]], axis=1)
    zg = jax.nn.sigmoid(jnp.dot(aggx, gxWl0[...]) + gxbl0[...]
                        + jnp.dot(rnn, gxWr0[...])
                        + jnp.dot(aggh, ghWl0[...]) + ghbl0[...]
                        + jnp.dot(h0v, ghWr0[...]))
    rg = jax.nn.sigmoid(jnp.dot(aggx, gxWl1[...]) + gxbl1[...]
                        + jnp.dot(rnn, gxWr1[...])
                        + jnp.dot(aggh, ghWl1[...]) + ghbl1[...]
                        + jnp.dot(h0v, ghWr1[...]))
    zg_o[...] = zg
    _store_split(rh_o, rg * h0v)
    pre_o[...] = (jnp.dot(aggx, gxWl2[...]) + gxbl2[...]
                  + jnp.dot(rnn, gxWr2[...]) + ghbl2[...])


def _tc5_body(p4, cntp, pre, rhs, zg, h0s, ghWl2, ghWr2, out_o):
    agg4 = _agg(p4, _invc(cntp))
    ht = jnp.tanh(pre[...] + jnp.dot(agg4, ghWl2[...])
                  + jnp.dot(_cat(rhs), ghWr2[...]))
    z = zg[...]
    out_o[...] = z * _cat(h0s) + (1.0 - z) * ht


def kernel(x, h, diff, edge_index, W_phi_x, b_phi_x, enc_Wl, enc_bl, enc_Wr,
           encm_Wl, encm_bl, encm_Wr, encs_Wl, encs_bl, encs_Wr,
           W_prior, b_prior, W_pm, b_pm, W_ps, b_ps, W_phi_z, b_phi_z,
           gx_Wl, gx_bl, gx_Wr, gh_Wl, gh_bl, gh_Wr):
    h0 = h[0]
    h0s = jnp.stack([h0[:, :_HH], h0[:, _HH:]])
    src3 = edge_index[0].astype(jnp.int32).reshape(_NS, _NCH, _CH)
    dst3 = edge_index[1].astype(jnp.int32).reshape(_NS, _NCH, _CH)
    eps1 = jax.random.normal(jax.random.key(7), (_N, _ZD), dtype=jnp.float32)
    r2 = lambda b: b.reshape(1, -1)
    segsum = _get_segsum()
    split_shape = jax.ShapeDtypeStruct((2, _N, _HH), jnp.float32)
    row_shape = lambda d: jax.ShapeDtypeStruct((_N, d), jnp.float32)

    phixs, prior_mean, prior_std = pl.pallas_call(
        _tc1_body,
        grid=(_GRID,),
        in_specs=[_row_spec(_XD), _split_spec(), _row_spec(1),
                  _w_spec(_XD, _HD), _w_spec(1, _HD),
                  _w_spec(_HD, _HD), _w_spec(1, _HD), _w_spec(1, _HD),
                  _w_spec(_HD, _ZD), _w_spec(1, _ZD),
                  _w_spec(_HD, _ZD), _w_spec(1, _ZD)],
        out_specs=[_split_spec(), _row_spec(_ZD), _row_spec(_ZD)],
        out_shape=[split_shape, row_shape(_ZD), row_shape(_ZD)],
    )(x, h0s, diff, W_phi_x, r2(b_phi_x), W_prior[:_HD], r2(W_prior[_HD]),
      r2(b_prior), W_pm, r2(b_pm), W_ps, r2(b_ps))

    p1x = segsum(phixs[None], src3, dst3)[0][:, 0]
    p1h = segsum(h0s[None], src3, dst3)[0][:, 0]
    (cntp,) = _get_segsum(0, True)(src3, dst3)

    encxs, agg1x, agg1h = pl.pallas_call(
        _tc2_body,
        grid=(_GRID,),
        in_specs=[_part_spec(), _part_spec(), _cnt_spec(),
                  _split_spec(), _split_spec(),
                  _w_spec(_HD, _HD), _w_spec(_HD, _HD), _w_spec(1, _HD),
                  _w_spec(_HD, _HD), _w_spec(_HD, _HD)],
        out_specs=[_split_spec(), _row_spec(_HD), _row_spec(_HD)],
        out_shape=[split_shape, row_shape(_HD), row_shape(_HD)],
    )(p1x, p1h, cntp, phixs, h0s,
      enc_Wl[:_HD], enc_Wl[_HD:], r2(enc_bl), enc_Wr[:_HD], enc_Wr[_HD:])

    p2 = segsum(encxs[None], src3, dst3)[0][:, 0]

    enc_x_mean, enc_x_std, z, phiz = pl.pallas_call(
        _tc3_body,
        grid=(_GRID,),
        in_specs=[_part_spec(), _cnt_spec(), _split_spec(),
                  _row_spec(_ZD),
                  _w_spec(_HD, _ZD), _w_spec(1, _ZD), _w_spec(_HD, _ZD),
                  _w_spec(_HD, _ZD), _w_spec(1, _ZD), _w_spec(_HD, _ZD),
                  _w_spec(_ZD, _HD), _w_spec(1, _HD)],
        out_specs=[_row_spec(_ZD), _row_spec(_ZD), _row_spec(_ZD),
                   _row_spec(_HD)],
        out_shape=[row_shape(_ZD), row_shape(_ZD), row_shape(_ZD),
                   row_shape(_HD)],
    )(p2, cntp, encxs, eps1, encm_Wl, r2(encm_bl), encm_Wr,
      encs_Wl, r2(encs_bl), encs_Wr, W_phi_z, r2(b_phi_z))

    phizs = jnp.stack([phiz[:, :_HH], phiz[:, _HH:]])
    p3 = segsum(phizs[None], src3, dst3)[0][:, 0]

    zg, rhs, pre = pl.pallas_call(
        _tc4_body,
        grid=(_GRID,),
        in_specs=[_part_spec(), _cnt_spec(), _row_spec(_HD),
                  _row_spec(_HD), _split_spec(), _split_spec(),
                  _row_spec(_HD)]
                 + [_w_spec(2 * _HD, _HD), _w_spec(1, _HD),
                    _w_spec(2 * _HD, _HD)] * 3
                 + [_w_spec(_HD, _HD), _w_spec(1, _HD),
                    _w_spec(_HD, _HD)] * 2
                 + [_w_spec(1, _HD)],
        out_specs=[_row_spec(_HD), _split_spec(), _row_spec(_HD)],
        out_shape=[row_shape(_HD), split_shape, row_shape(_HD)],
    )(p3, cntp, agg1x, agg1h, phixs, h0s, phiz,
      gx_Wl[0], r2(gx_bl[0]), gx_Wr[0],
      gx_Wl[1], r2(gx_bl[1]), gx_Wr[1],
      gx_Wl[2], r2(gx_bl[2]), gx_Wr[2],
      gh_Wl[0], r2(gh_bl[0]), gh_Wr[0],
      gh_Wl[1], r2(gh_bl[1]), gh_Wr[1],
      r2(gh_bl[2]))

    p4 = segsum(rhs[None], src3, dst3)[0][:, 0]

    out = pl.pallas_call(
        _tc5_body,
        grid=(_GRID,),
        in_specs=[_part_spec(), _cnt_spec(), _row_spec(_HD),
                  _split_spec(), _row_spec(_HD), _split_spec(),
                  _w_spec(_HD, _HD), _w_spec(_HD, _HD)],
        out_specs=[_row_spec(_HD)],
        out_shape=[row_shape(_HD)],
    )(p4, cntp, pre, rhs, zg, h0s, gh_Wl[2], gh_Wr[2])[0]

    return (prior_mean, prior_std, enc_x_mean, enc_x_std, z, out[None])


# speedup vs baseline: 1.1318x; 1.0073x over previous
import functools
import jax
import jax.numpy as jnp
from jax import lax
from jax.experimental import pallas as pl
from jax.experimental.pallas import tpu as pltpu
from jax.experimental.pallas import tpu_sc as plsc

_N = 10000
_E = 320000
_XD = 128
_HD = 128
_ZD = 64
_HH = 64

_NC = 2
_NS = 16
_EPW = _E // _NS
_CH = 80
_NCH = _EPW // _CH
_K = 4
_NP = 10112
_NPS = _NP // _NS
_ZR = 128
_STAGE = [(o, min(_ZR, _NPS - o)) for o in range(0, _NPS, _ZR)]

_BN = 2000
_GRID = _N // _BN


def _fill_vmem(ref, nrows, ncols, value):
    vec = jnp.full((16,), value, jnp.float32)

    def row(r, _):
        def col(j, _):
            ref[r, pl.ds(j * 16, 16)] = vec
            return 0
        return lax.fori_loop(0, ncols // 16, col, 0)

    lax.fori_loop(0, nrows, row, 0)


def _make_segsum(n_tables, with_count):
    outs = []
    if n_tables:
        outs.append(jax.ShapeDtypeStruct((_NC, n_tables, _NP, _HH),
                                         jnp.float32))
    if with_count:
        outs.append(jax.ShapeDtypeStruct((1, _NP, 16), jnp.float32))
    scratch = [
        pltpu.VMEM((_NCH, _CH), jnp.int32),
        pltpu.VMEM((_NCH, _CH), jnp.int32),
        pltpu.SemaphoreType.DMA,
        pltpu.SemaphoreType.DMA,
    ]
    if n_tables:
        scratch += [
            pltpu.VMEM((_K, _CH, _HH), jnp.float32),
            pltpu.VMEM((_ZR, _HH), jnp.float32),
            pltpu.VMEM_SHARED((_NP, _HH), jnp.float32),
        ]
    if with_count:
        scratch += [
            pltpu.VMEM((_CH, 16), jnp.float32),
            pltpu.VMEM((_ZR, 16), jnp.float32),
            pltpu.VMEM_SHARED((_NP, 16), jnp.float32),
        ]
    mesh = plsc.VectorSubcoreMesh(core_axis_name="c", subcore_axis_name="s",
                                  num_cores=_NC, num_subcores=_NS)

    @functools.partial(pl.kernel, out_type=tuple(outs), mesh=mesh,
                       scratch_types=scratch,
                       compiler_params=pltpu.CompilerParams(
                           use_tc_tiling_on_sc=False))
    def seg(*refs):
        if n_tables:
            table = refs[0]
            src3, dst3 = refs[1], refs[2]
            out = refs[3]
            o = 4
        else:
            src3, dst3 = refs[0], refs[1]
            o = 2
        if with_count:
            cnt_out = refs[o]
            o += 1
        src_v, dst_v, gsem, ssem = refs[o:o + 4]
        o += 4
        if n_tables:
            rows_v, zbuf, acc = refs[o:o + 3]
            o += 3
        if with_count:
            ones_v, cbuf, cacc = refs[o:o + 3]
        c = lax.axis_index("c")
        s = lax.axis_index("s")
        r0 = s * _NPS

        if n_tables:
            _fill_vmem(zbuf, _ZR, _HH, 0.0)
            pltpu.sync_copy(src3.at[s], src_v)
        pltpu.sync_copy(dst3.at[s], dst_v)

        def half_pass(tab, out_slice):
            for st, sz in _STAGE:
                pltpu.sync_copy(zbuf.at[pl.ds(0, sz)],
                                acc.at[pl.ds(r0 + st, sz)])
            plsc.subcore_barrier()

            pltpu.async_copy(tab.at[src_v.at[0]], rows_v.at[0], gsem)
            pltpu.async_copy(tab.at[src_v.at[1]], rows_v.at[1], gsem)

            def chunk(j, _):
                @pl.when(j >= _K - 2)
                def _():
                    pltpu.make_async_copy(rows_v.at[0], acc.at[dst_v.at[0]],
                                          ssem).wait()

                @pl.when(j + 2 < _NCH)
                def _():
                    pltpu.async_copy(tab.at[src_v.at[j + 2]],
                                     rows_v.at[(j + 2) % _K], gsem)

                pltpu.make_async_copy(tab.at[src_v.at[j]],
                                      rows_v.at[j % _K], gsem).wait()
                pltpu.async_copy(rows_v.at[j % _K], acc.at[dst_v.at[j]],
                                 ssem, add=True)
                return 0

            lax.fori_loop(0, _NCH, chunk, 0)
            for _tail in range(_K - 2):
                pltpu.make_async_copy(rows_v.at[0], acc.at[dst_v.at[0]],
                                      ssem).wait()
            plsc.subcore_barrier()

            for st, sz in _STAGE:
                pltpu.sync_copy(acc.at[pl.ds(r0 + st, sz)],
                                zbuf.at[pl.ds(0, sz)])
                pltpu.sync_copy(zbuf.at[pl.ds(0, sz)],
                                out_slice.at[pl.ds(r0 + st, sz)])
            _fill_vmem(zbuf, _ZR, _HH, 0.0)
            plsc.subcore_barrier()

        if n_tables:
            def pass_body(t, _):
                half_pass(table.at[2 * t + c], out.at[c].at[t])
                return 0

            lax.fori_loop(0, n_tables, pass_body, 0)

        if with_count:
            _fill_vmem(cbuf, _ZR, 16, 0.0)
            _fill_vmem(ones_v, _CH, 16, 1.0)
            for st, sz in _STAGE:
                pltpu.sync_copy(cbuf.at[pl.ds(0, sz)],
                                cacc.at[pl.ds(r0 + st, sz)])
            plsc.subcore_barrier()

            def cgroup(j, _):
                @pl.when(j >= 2)
                def _():
                    pltpu.make_async_copy(ones_v, cacc.at[dst_v.at[0]],
                                          ssem).wait()

                pltpu.async_copy(ones_v, cacc.at[dst_v.at[j]], ssem, add=True)
                return 0

            lax.fori_loop(0, _NCH, cgroup, 0)
            for _tail in range(2):
                pltpu.make_async_copy(ones_v, cacc.at[dst_v.at[0]],
                                      ssem).wait()
            plsc.subcore_barrier()
            @pl.when(c == 0)
            def _():
                for st, sz in _STAGE:
                    pltpu.sync_copy(cacc.at[pl.ds(r0 + st, sz)],
                                    cbuf.at[pl.ds(0, sz)])
                    pltpu.sync_copy(cbuf.at[pl.ds(0, sz)],
                                    cnt_out.at[0].at[pl.ds(r0 + st, sz)])

    return seg


@functools.lru_cache(maxsize=None)
def _get_segsum(n_tables=1, with_count=False):
    return _make_segsum(n_tables, with_count)


def _row_spec(d):
    return pl.BlockSpec((_BN, d), lambda i: (i, 0))


def _split_spec():
    return pl.BlockSpec((2, _BN, _HH), lambda i: (0, i, 0))


def _part_spec():
    return pl.BlockSpec((_NC, _BN, _HH), lambda i: (0, i, 0))


def _cnt_spec():
    return pl.BlockSpec((1, _BN, 16), lambda i: (0, i, 0))


def _w_spec(r, c):
    return pl.BlockSpec((r, c), lambda i: (0, 0))


def _split_out(d2):
    return jax.ShapeDtypeStruct((2, _N, _HH), jnp.float32)


def _invc(cntp_ref):
    return 1.0 / jnp.maximum(cntp_ref[0, :, 0:1], 1.0)


def _agg(p_ref, ic):
    return jnp.concatenate([p_ref[0], p_ref[1]], axis=1) * ic


def _cat(sp_ref):
    return jnp.concatenate([sp_ref[0], sp_ref[1]], axis=1)


def _store_split(out_ref, val):
    out_ref[0] = val[:, :_HH]
    out_ref[1] = val[:, _HH:]


def _tc1_body(x, h0s, diff, Wx, bx, Wph, wpd, bp, Wpm, bpm, Wps, bps,
              phix_o, pm_o, ps_o):
    _store_split(phix_o, jax.nn.relu(jnp.dot(x[...], Wx[...]) + bx[...]))
    prior = jax.nn.relu(jnp.dot(_cat(h0s), Wph[...]) + diff[...] * wpd[...]
                        + bp[...])
    pm_o[...] = jnp.dot(prior, Wpm[...]) + bpm[...]
    ps_o[...] = jax.nn.softplus(jnp.dot(prior, Wps[...]) + bps[...])


def _tc2_body(p1x, p1h, cntp, phixs, h0s, Wlt, Wlb, bl, Wrt, Wrb,
              encx_o, agg1x_o, agg1h_o):
    ic = _invc(cntp)
    agg1x = _agg(p1x, ic)
    agg1h = _agg(p1h, ic)
    agg1x_o[...] = agg1x
    agg1h_o[...] = agg1h
    _store_split(encx_o, jax.nn.relu(
        jnp.dot(agg1x, Wlt[...]) + jnp.dot(agg1h, Wlb[...]) + bl[...]
        + jnp.dot(_cat(phixs), Wrt[...]) + jnp.dot(_cat(h0s), Wrb[...])))


def _tc3_body(p2, cntp, encxs, eps, Wml, bml, Wmr, Wsl, bsl, Wsr, Wz, bz,
              mean_o, std_o, z_o, phiz_o):
    agg2 = _agg(p2, _invc(cntp))
    ex = _cat(encxs)
    mean = jnp.dot(agg2, Wml[...]) + bml[...] + jnp.dot(ex, Wmr[...])
    std = jax.nn.softplus(jnp.dot(agg2, Wsl[...]) + bsl[...]
                          + jnp.dot(ex, Wsr[...]))
    zz = eps[...] * std + mean
    mean_o[...] = mean
    std_o[...] = std
    z_o[...] = zz
    phiz_o[...] = jax.nn.relu(jnp.dot(zz, Wz[...]) + bz[...])


def _tc4_body(p3, cntp, agg1x, agg1h, phixs, h0s, phiz,
              gxWl0, gxbl0, gxWr0, gxWl1, gxbl1, gxWr1, gxWl2, gxbl2, gxWr2,
              ghWl0, ghbl0, ghWr0, ghWl1, ghbl1, ghWr1, ghbl2,
              zg_o, rh_o, pre_o):
    agg3 = _agg(p3, _invc(cntp))
    aggx = jnp.concatenate([agg1x[...], agg3], axis=1)
    aggh = agg1h[...]
    h0v = _cat(h0s)
    rnn = jnp.concatenate([_cat(phixs), phiz[...]], axis=1)
    zg = jax.nn.sigmoid(jnp.dot(aggx, gxWl0[...]) + gxbl0[...]
                        + jnp.dot(rnn, gxWr0[...])
                        + jnp.dot(aggh, ghWl0[...]) + ghbl0[...]
                        + jnp.dot(h0v, ghWr0[...]))
    rg = jax.nn.sigmoid(jnp.dot(aggx, gxWl1[...]) + gxbl1[...]
                        + jnp.dot(rnn, gxWr1[...])
                        + jnp.dot(aggh, ghWl1[...]) + ghbl1[...]
                        + jnp.dot(h0v, ghWr1[...]))
    zg_o[...] = zg
    _store_split(rh_o, rg * h0v)
    pre_o[...] = (jnp.dot(aggx, gxWl2[...]) + gxbl2[...]
                  + jnp.dot(rnn, gxWr2[...]) + ghbl2[...])


def _tc5_body(p4, cntp, pre, rhs, zg, h0s, ghWl2, ghWr2, out_o):
    agg4 = _agg(p4, _invc(cntp))
    ht = jnp.tanh(pre[...] + jnp.dot(agg4, ghWl2[...])
                  + jnp.dot(_cat(rhs), ghWr2[...]))
    z = zg[...]
    out_o[...] = z * _cat(h0s) + (1.0 - z) * ht


def kernel(x, h, diff, edge_index, W_phi_x, b_phi_x, enc_Wl, enc_bl, enc_Wr,
           encm_Wl, encm_bl, encm_Wr, encs_Wl, encs_bl, encs_Wr,
           W_prior, b_prior, W_pm, b_pm, W_ps, b_ps, W_phi_z, b_phi_z,
           gx_Wl, gx_bl, gx_Wr, gh_Wl, gh_bl, gh_Wr):
    h0 = h[0]
    h0s = jnp.stack([h0[:, :_HH], h0[:, _HH:]])
    src3 = edge_index[0].astype(jnp.int32).reshape(_NS, _NCH, _CH)
    dst3 = edge_index[1].astype(jnp.int32).reshape(_NS, _NCH, _CH)
    eps1 = jax.random.normal(jax.random.key(7), (_N, _ZD), dtype=jnp.float32)
    r2 = lambda b: b.reshape(1, -1)
    segsum = _get_segsum()
    split_shape = jax.ShapeDtypeStruct((2, _N, _HH), jnp.float32)
    row_shape = lambda d: jax.ShapeDtypeStruct((_N, d), jnp.float32)

    phixs, prior_mean, prior_std = pl.pallas_call(
        _tc1_body,
        grid=(_GRID,),
        in_specs=[_row_spec(_XD), _split_spec(), _row_spec(1),
                  _w_spec(_XD, _HD), _w_spec(1, _HD),
                  _w_spec(_HD, _HD), _w_spec(1, _HD), _w_spec(1, _HD),
                  _w_spec(_HD, _ZD), _w_spec(1, _ZD),
                  _w_spec(_HD, _ZD), _w_spec(1, _ZD)],
        out_specs=[_split_spec(), _row_spec(_ZD), _row_spec(_ZD)],
        out_shape=[split_shape, row_shape(_ZD), row_shape(_ZD)],
    )(x, h0s, diff, W_phi_x, r2(b_phi_x), W_prior[:_HD], r2(W_prior[_HD]),
      r2(b_prior), W_pm, r2(b_pm), W_ps, r2(b_ps))

    p1x = segsum(phixs, src3, dst3)[0][:, 0]
    p1h = segsum(h0s, src3, dst3)[0][:, 0]
    (cntp,) = _get_segsum(0, True)(src3, dst3)

    encxs, agg1x, agg1h = pl.pallas_call(
        _tc2_body,
        grid=(_GRID,),
        in_specs=[_part_spec(), _part_spec(), _cnt_spec(),
                  _split_spec(), _split_spec(),
                  _w_spec(_HD, _HD), _w_spec(_HD, _HD), _w_spec(1, _HD),
                  _w_spec(_HD, _HD), _w_spec(_HD, _HD)],
        out_specs=[_split_spec(), _row_spec(_HD), _row_spec(_HD)],
        out_shape=[split_shape, row_shape(_HD), row_shape(_HD)],
    )(p1x, p1h, cntp, phixs, h0s,
      enc_Wl[:_HD], enc_Wl[_HD:], r2(enc_bl), enc_Wr[:_HD], enc_Wr[_HD:])

    p2 = segsum(encxs, src3, dst3)[0][:, 0]

    enc_x_mean, enc_x_std, z, phiz = pl.pallas_call(
        _tc3_body,
        grid=(_GRID,),
        in_specs=[_part_spec(), _cnt_spec(), _split_spec(),
                  _row_spec(_ZD),
                  _w_spec(_HD, _ZD), _w_spec(1, _ZD), _w_spec(_HD, _ZD),
                  _w_spec(_HD, _ZD), _w_spec(1, _ZD), _w_spec(_HD, _ZD),
                  _w_spec(_ZD, _HD), _w_spec(1, _HD)],
        out_specs=[_row_spec(_ZD), _row_spec(_ZD), _row_spec(_ZD),
                   _row_spec(_HD)],
        out_shape=[row_shape(_ZD), row_shape(_ZD), row_shape(_ZD),
                   row_shape(_HD)],
    )(p2, cntp, encxs, eps1, encm_Wl, r2(encm_bl), encm_Wr,
      encs_Wl, r2(encs_bl), encs_Wr, W_phi_z, r2(b_phi_z))

    phizs = jnp.stack([phiz[:, :_HH], phiz[:, _HH:]])
    p3 = segsum(phizs, src3, dst3)[0][:, 0]

    zg, rhs, pre = pl.pallas_call(
        _tc4_body,
        grid=(_GRID,),
        in_specs=[_part_spec(), _cnt_spec(), _row_spec(_HD),
                  _row_spec(_HD), _split_spec(), _split_spec(),
                  _row_spec(_HD)]
                 + [_w_spec(2 * _HD, _HD), _w_spec(1, _HD),
                    _w_spec(2 * _HD, _HD)] * 3
                 + [_w_spec(_HD, _HD), _w_spec(1, _HD),
                    _w_spec(_HD, _HD)] * 2
                 + [_w_spec(1, _HD)],
        out_specs=[_row_spec(_HD), _split_spec(), _row_spec(_HD)],
        out_shape=[row_shape(_HD), split_shape, row_shape(_HD)],
    )(p3, cntp, agg1x, agg1h, phixs, h0s, phiz,
      gx_Wl[0], r2(gx_bl[0]), gx_Wr[0],
      gx_Wl[1], r2(gx_bl[1]), gx_Wr[1],
      gx_Wl[2], r2(gx_bl[2]), gx_Wr[2],
      gh_Wl[0], r2(gh_bl[0]), gh_Wr[0],
      gh_Wl[1], r2(gh_bl[1]), gh_Wr[1],
      r2(gh_bl[2]))

    p4 = segsum(rhs, src3, dst3)[0][:, 0]

    out = pl.pallas_call(
        _tc5_body,
        grid=(_GRID,),
        in_specs=[_part_spec(), _cnt_spec(), _row_spec(_HD),
                  _split_spec(), _row_spec(_HD), _split_spec(),
                  _w_spec(_HD, _HD), _w_spec(_HD, _HD)],
        out_specs=[_row_spec(_HD)],
        out_shape=[row_shape(_HD)],
    )(p4, cntp, pre, rhs, zg, h0s, gh_Wl[2], gh_Wr[2])[0]

    return (prior_mean, prior_std, enc_x_mean, enc_x_std, z, out[None])
